# baseline (speedup 1.0000x reference)
import functools

import jax
import jax.numpy as jnp
import numpy as np
from jax import lax
from jax.experimental import pallas as pl
from jax.experimental.pallas import tpu as pltpu
from jax.experimental.pallas import tpu_sc as plsc

_BN = float(1.0 / np.sqrt(1.0 + 1e-5))
_KNN = 10


def _conv_body(x_ref, w_ref, b_ref, o_ref):
    x = x_ref[0]
    n = x.shape[1]
    sh = (
        jnp.concatenate([x[:, :1], x[:, :1], x[:, : n - 2]], axis=1),
        jnp.concatenate([x[:, :1], x[:, : n - 1]], axis=1),
        x,
        jnp.concatenate([x[:, 1:], x[:, n - 1:]], axis=1),
        jnp.concatenate([x[:, 2:], x[:, n - 1:], x[:, n - 1:]], axis=1),
    )
    y = b_ref[...]
    for t in range(5):
        y = y + jnp.dot(w_ref[t], sh[t])
    o_ref[0] = jax.nn.gelu(y * _BN)


def _fuse2_body(z_ref, dn_ref, sp_ref, ws_ref, wd_ref, bd_ref,
                wdd_ref, wr_ref, bs_ref, us_ref, ud_ref):
    z = z_ref[0]
    sp = sp_ref[0]
    d = jnp.max(z, axis=-1)
    us = jax.nn.gelu(
        (jnp.dot(ws_ref[...], sp) + jnp.dot(wd_ref[...], d) + bd_ref[...]) * _BN)
    us_ref[0] = us
    r = jnp.dot(wr_ref[...], sp)
    dn = dn_ref[0]
    s = dn.shape[1]
    for si in range(s):
        col = jax.nn.gelu(
            (jnp.dot(wdd_ref[...], dn[:, si, :]) + r[:, si:si + 1] + bs_ref[...]) * _BN)
        ud_ref[0, :, si, :] = col


def _knn_body(xb_ref, xa_ref, o_ref, *, n, k):
    xb = xb_ref[0]
    xa = xa_ref[0]
    sqb = jnp.sum(xb * xb, axis=1, keepdims=True)
    sqa = jnp.sum(xa * xa, axis=1)
    cross = lax.dot_general(xb, xa, (((1,), (1,)), ((), ())))
    dist = sqb + sqa[None, :] - 2.0 * cross
    iota = lax.broadcasted_iota(jnp.int32, dist.shape, 1)
    base = pl.program_id(0) * n
    cols = []
    for _ in range(k):
        j = jnp.argmin(dist, axis=1).astype(jnp.int32)
        cols.append(j)
        dist = jnp.where(iota == j[:, None], jnp.inf, dist)
    idx = jnp.stack(cols, axis=1)
    o_ref[0] = idx + base


def _edge_body(nb_ref, cen_ref, w1a_ref, w1b_ref, b1_ref, w2_ref, b2_ref,
               o_ref, *, k, hp):
    nb = nb_ref[...]
    cen = cen_ref[0]
    r, c = cen.shape
    h1n = w1a_ref.shape[1]
    cenr = jnp.broadcast_to(cen[:, None, :], (r, k, c)).reshape(r * k, c)
    e = nb - cenr
    c1 = jnp.dot(cen, w1b_ref[...]) + b1_ref[...]
    c1r = jnp.broadcast_to(c1[:, None, :], (r, k, h1n)).reshape(r * k, h1n)
    h1 = jax.nn.gelu((jnp.dot(e, w1a_ref[...]) + c1r) * _BN)
    h2 = jax.nn.gelu((jnp.dot(h1, w2_ref[...]) + b2_ref[...]) * _BN)
    h2d = h2.shape[1]
    xo = jnp.max(h2.reshape(r, k, h2d), axis=1)
    if hp > h2d:
        xo = jnp.concatenate(
            [xo, jnp.zeros((r, hp - h2d), xo.dtype)], axis=1)
    o_ref[0] = xo


def _pw_body(x1_ref, x2_ref, wa_ref, wb_ref, b1_ref, w2_ref, b2_ref, o_ref):
    h = jax.nn.gelu(
        (jnp.dot(x1_ref[0], wa_ref[...]) + jnp.dot(x2_ref[0], wb_ref[...])
         + b1_ref[...]) * _BN)
    o_ref[0] = jax.nn.gelu((jnp.dot(h, w2_ref[...]) + b2_ref[...]) * _BN)


def _make_sc_gather(n_rows, d, b_total):
    info = plsc.get_sparse_core_info()
    nc, ns = info.num_cores, info.num_subcores
    nw = nc * ns
    assert b_total % (8 * nw) == 0, (b_total, nw)
    b_per_w = b_total // nw
    ch = min(b_per_w, 128)
    assert b_per_w % ch == 0
    n_chunks = b_per_w // ch
    mesh = plsc.VectorSubcoreMesh(core_axis_name="c", subcore_axis_name="s")

    if n_chunks < 2:
        @functools.partial(
            pl.kernel, mesh=mesh,
            out_type=jax.ShapeDtypeStruct((b_total, d), jnp.float32),
            scratch_types=[
                pltpu.VMEM((ch,), jnp.int32),
                pltpu.VMEM((ch, d), jnp.float32),
                pltpu.SemaphoreType.DMA,
            ])
        def gath(table_hbm, idx_hbm, out_hbm, idx_v, rows_v, sem):
            wid = lax.axis_index("s") * nc + lax.axis_index("c")
            base = wid * b_per_w
            pltpu.sync_copy(idx_hbm.at[pl.ds(base, ch)], idx_v)
            pltpu.async_copy(table_hbm.at[idx_v], rows_v, sem).wait()
            pltpu.sync_copy(rows_v, out_hbm.at[pl.ds(base, ch)])
        return gath

    assert n_chunks % 2 == 0

    @functools.partial(
        pl.kernel, mesh=mesh,
        out_type=jax.ShapeDtypeStruct((b_total, d), jnp.float32),
        scratch_types=[
            pltpu.VMEM((b_per_w,), jnp.int32),
            pltpu.VMEM((ch, d), jnp.float32),
            pltpu.VMEM((ch, d), jnp.float32),
            pltpu.SemaphoreType.DMA,
            pltpu.SemaphoreType.DMA,
            pltpu.SemaphoreType.DMA,
            pltpu.SemaphoreType.DMA,
        ])
    def gath(table_hbm, idx_hbm, out_hbm, idx_v, rows0, rows1,
             sg0, sg1, sw0, sw1):
        wid = lax.axis_index("s") * nc + lax.axis_index("c")
        base = wid * b_per_w
        pltpu.sync_copy(idx_hbm.at[pl.ds(base, b_per_w)], idx_v)

        def body(i, carry):
            o0 = 2 * i * ch
            o1 = o0 + ch

            @pl.when(i > 0)
            def _():
                pltpu.make_async_copy(
                    rows0, out_hbm.at[pl.ds(0, ch)], sw0).wait()
                pltpu.make_async_copy(
                    rows1, out_hbm.at[pl.ds(0, ch)], sw1).wait()

            g0 = pltpu.async_copy(
                table_hbm.at[idx_v.at[pl.ds(o0, ch)]], rows0, sg0)
            g1 = pltpu.async_copy(
                table_hbm.at[idx_v.at[pl.ds(o1, ch)]], rows1, sg1)
            g0.wait()
            pltpu.async_copy(rows0, out_hbm.at[pl.ds(base + o0, ch)], sw0)
            g1.wait()
            pltpu.async_copy(rows1, out_hbm.at[pl.ds(base + o1, ch)], sw1)
            return carry

        lax.fori_loop(0, n_chunks // 2, body, 0)
        pltpu.make_async_copy(rows0, out_hbm.at[pl.ds(0, ch)], sw0).wait()
        pltpu.make_async_copy(rows1, out_hbm.at[pl.ds(0, ch)], sw1).wait()

    return gath


def _gather_rows(table, idx):
    n_rows, d = table.shape
    return _make_sc_gather(n_rows, d, idx.shape[0])(table, idx)


def _knn_call(x_n, r_blk):
    b, n, c = x_n.shape
    body = functools.partial(_knn_body, n=n, k=_KNN)
    return pl.pallas_call(
        body,
        grid=(b, n // r_blk),
        in_specs=[
            pl.BlockSpec((1, r_blk, c), lambda i, j: (i, j, 0)),
            pl.BlockSpec((1, n, c), lambda i, j: (i, 0, 0)),
        ],
        out_specs=pl.BlockSpec((1, r_blk, _KNN), lambda i, j: (i, j, 0)),
        out_shape=jax.ShapeDtypeStruct((b, n, _KNN), jnp.int32),
    )(x_n, x_n)


def _edge_call(nb, x_n, w1, b1, w2, b2, r_blk, hp):
    b, n, c = x_n.shape
    h1n = w1.shape[0]
    h2n = w2.shape[0]
    w1a = jnp.transpose(w1[:, :c])
    w1b = jnp.transpose(w1[:, c:])
    if w1b.shape[0] < c:
        w1b = jnp.pad(w1b, ((0, c - w1b.shape[0]), (0, 0)))
    w2t = jnp.transpose(w2)
    nblk = n // r_blk
    body = functools.partial(_edge_body, k=_KNN, hp=hp)
    return pl.pallas_call(
        body,
        grid=(b, nblk),
        in_specs=[
            pl.BlockSpec((r_blk * _KNN, c), lambda i, j: (i * nblk + j, 0)),
            pl.BlockSpec((1, r_blk, c), lambda i, j: (i, j, 0)),
            pl.BlockSpec((c, h1n), lambda i, j: (0, 0)),
            pl.BlockSpec((c, h1n), lambda i, j: (0, 0)),
            pl.BlockSpec((1, h1n), lambda i, j: (0, 0)),
            pl.BlockSpec((h1n, h2n), lambda i, j: (0, 0)),
            pl.BlockSpec((1, h2n), lambda i, j: (0, 0)),
        ],
        out_specs=pl.BlockSpec((1, r_blk, hp), lambda i, j: (i, j, 0)),
        out_shape=jax.ShapeDtypeStruct((b, n, hp), jnp.float32),
    )(nb, x_n, w1a, w1b, b1.reshape(1, h1n), w2t, b2.reshape(1, h2n))


def _pw_call(x1, x2, w1, b1, w2, b2, r_blk, d1_real):
    b, n, c1p = x1.shape
    c2 = x2.shape[2]
    h1n = w1.shape[0]
    h2n = w2.shape[0]
    wa = jnp.transpose(w1[:, :d1_real])
    wa = jnp.pad(wa, ((0, c1p - d1_real), (0, 0)))
    wb = jnp.transpose(w1[:, d1_real:])
    w2t = jnp.transpose(w2)
    nblk = n // r_blk
    return pl.pallas_call(
        _pw_body,
        grid=(b, nblk),
        in_specs=[
            pl.BlockSpec((1, r_blk, c1p), lambda i, j: (i, j, 0)),
            pl.BlockSpec((1, r_blk, c2), lambda i, j: (i, j, 0)),
            pl.BlockSpec((c1p, h1n), lambda i, j: (0, 0)),
            pl.BlockSpec((c2, h1n), lambda i, j: (0, 0)),
            pl.BlockSpec((1, h1n), lambda i, j: (0, 0)),
            pl.BlockSpec((h1n, h2n), lambda i, j: (0, 0)),
            pl.BlockSpec((1, h2n), lambda i, j: (0, 0)),
        ],
        out_specs=pl.BlockSpec((1, r_blk, h2n), lambda i, j: (i, j, 0)),
        out_shape=jax.ShapeDtypeStruct((b, n, h2n), jnp.float32),
    )(x1, x2, wa, wb, b1.reshape(1, h1n), w2t, b2.reshape(1, h2n))


def _gcn(x_n, gp, r_blk):
    b, n, c = x_n.shape
    (w11, b11), (w12, b12) = gp['c1']
    (w21, b21), (w22, b22) = gp['c2']
    (w31, b31), (w32, b32) = gp['c3']
    h2a = w12.shape[0]
    h2a_p = -(-h2a // 128) * 128

    knn_r = min(n, 1024)
    idx1 = _knn_call(x_n, knn_r)
    nb1 = _gather_rows(x_n.reshape(b * n, c), idx1.reshape(-1))
    x1 = _edge_call(nb1, x_n, w11, b11, w12, b12, r_blk, h2a_p)

    w21p = jnp.pad(w21, ((0, 0), (0, 2 * h2a_p - w21.shape[1])))
    w21a = jnp.pad(w21[:, :h2a], ((0, 0), (0, h2a_p - h2a)))
    w21b = jnp.pad(w21[:, h2a:], ((0, 0), (0, h2a_p - h2a)))
    w21ab = jnp.concatenate([w21a, w21b], axis=1)
    del w21p
    idx2 = _knn_call(x1, knn_r)
    nb2 = _gather_rows(x1.reshape(b * n, h2a_p), idx2.reshape(-1))
    x2 = _edge_call(nb2, x1, w21ab, b21, w22, b22, r_blk, w22.shape[0])

    return _pw_call(x1, x2, w31, b31, w32, b32, r_blk, h2a)


def kernel(sparse_fea, dense_fea, params):
    p = params
    b, c, s, sp_ = dense_fea.shape
    nd = s * sp_
    ns = sparse_fea.shape[2]

    tw = jnp.transpose(p['d2s_tW'], (2, 0, 1))
    tb = p['d2s_tb'].reshape(c, 1)
    d2 = dense_fea.reshape(b, c, nd)
    z = pl.pallas_call(
        _conv_body,
        grid=(b,),
        in_specs=[
            pl.BlockSpec((1, c, nd), lambda i: (i, 0, 0)),
            pl.BlockSpec((5, c, c), lambda i: (0, 0, 0)),
            pl.BlockSpec((c, 1), lambda i: (0, 0)),
        ],
        out_specs=pl.BlockSpec((1, c, nd), lambda i: (i, 0, 0)),
        out_shape=jax.ShapeDtypeStruct((b, c, nd), jnp.float32),
    )(d2, tw, tb)
    z4 = z.reshape(b, c, s, sp_)

    csp = sparse_fea.shape[1]
    ws = p['d2s_W'][:, :csp]
    wd = p['d2s_W'][:, csp:]
    wdd = p['s2d_W'][:, :c]
    wr = p['s2d_W'][:, c:]
    us, ud = pl.pallas_call(
        _fuse2_body,
        grid=(b,),
        in_specs=[
            pl.BlockSpec((1, c, s, sp_), lambda i: (i, 0, 0, 0)),
            pl.BlockSpec((1, c, s, sp_), lambda i: (i, 0, 0, 0)),
            pl.BlockSpec((1, csp, ns), lambda i: (i, 0, 0)),
            pl.BlockSpec(ws.shape, lambda i: (0, 0)),
            pl.BlockSpec(wd.shape, lambda i: (0, 0)),
            pl.BlockSpec((csp, 1), lambda i: (0, 0)),
            pl.BlockSpec(wdd.shape, lambda i: (0, 0)),
            pl.BlockSpec(wr.shape, lambda i: (0, 0)),
            pl.BlockSpec((c, 1), lambda i: (0, 0)),
        ],
        out_specs=[
            pl.BlockSpec((1, csp, ns), lambda i: (i, 0, 0)),
            pl.BlockSpec((1, c, s, sp_), lambda i: (i, 0, 0, 0)),
        ],
        out_shape=[
            jax.ShapeDtypeStruct((b, csp, ns), jnp.float32),
            jax.ShapeDtypeStruct((b, c, s, sp_), jnp.float32),
        ],
    )(z4, dense_fea, sparse_fea, ws, wd, p['d2s_b'].reshape(csp, 1),
      wdd, wr, p['s2d_b'].reshape(c, 1))

    us_n = jnp.transpose(us, (0, 2, 1))
    xd_n = jnp.transpose(ud.reshape(b, c, nd), (0, 2, 1))

    sp_out = _gcn(us_n, p['sp_gcn'], r_blk=ns)
    bh = b // 2 if b % 2 == 0 else b
    dn_out = jnp.concatenate(
        [_gcn(xd_n[i:i + bh], p['dn_gcn'], r_blk=256)
         for i in range(0, b, bh)], axis=0)

    sparse_out = jnp.transpose(sp_out, (0, 2, 1))
    dense_out = jnp.transpose(dn_out, (0, 2, 1)).reshape(b, -1, s, sp_)
    return sparse_out, dense_out


# speedup vs baseline: 1.0188x; 1.0188x over previous
import functools

import jax
import jax.numpy as jnp
import numpy as np
from jax import lax
from jax.experimental import pallas as pl
from jax.experimental.pallas import tpu as pltpu
from jax.experimental.pallas import tpu_sc as plsc

_BN = float(1.0 / np.sqrt(1.0 + 1e-5))
_KNN = 10


def _conv_body(x_ref, w_ref, b_ref, o_ref):
    x = x_ref[0]
    n = x.shape[1]
    sh = (
        jnp.concatenate([x[:, :1], x[:, :1], x[:, : n - 2]], axis=1),
        jnp.concatenate([x[:, :1], x[:, : n - 1]], axis=1),
        x,
        jnp.concatenate([x[:, 1:], x[:, n - 1:]], axis=1),
        jnp.concatenate([x[:, 2:], x[:, n - 1:], x[:, n - 1:]], axis=1),
    )
    y = b_ref[...]
    for t in range(5):
        y = y + jnp.dot(w_ref[t], sh[t])
    o_ref[0] = jax.nn.gelu(y * _BN)


def _fuse2_body(z_ref, dn_ref, sp_ref, ws_ref, wd_ref, bd_ref,
                wdd_ref, wr_ref, bs_ref, us_ref, ud_ref):
    z = z_ref[0]
    sp = sp_ref[0]
    d = jnp.max(z, axis=-1)
    us = jax.nn.gelu(
        (jnp.dot(ws_ref[...], sp) + jnp.dot(wd_ref[...], d) + bd_ref[...]) * _BN)
    us_ref[0] = us
    r = jnp.dot(wr_ref[...], sp)
    dn = dn_ref[0]
    s = dn.shape[1]
    for si in range(s):
        col = jax.nn.gelu(
            (jnp.dot(wdd_ref[...], dn[:, si, :]) + r[:, si:si + 1] + bs_ref[...]) * _BN)
        ud_ref[0, :, si, :] = col


def _knn_body(xb_ref, xa_ref, o_ref, *, n, k):
    xb = xb_ref[0]
    xa = xa_ref[0]
    sqb = jnp.sum(xb * xb, axis=1, keepdims=True)
    sqa = jnp.sum(xa * xa, axis=1)
    cross = lax.dot_general(xb, xa, (((1,), (1,)), ((), ())))
    dist = sqb + sqa[None, :] - 2.0 * cross
    iota = lax.broadcasted_iota(jnp.int32, dist.shape, 1)
    base = pl.program_id(0) * n
    cols = []
    for _ in range(k):
        j = jnp.argmin(dist, axis=1).astype(jnp.int32)
        cols.append(j)
        dist = jnp.where(iota == j[:, None], jnp.inf, dist)
    idx = jnp.stack(cols, axis=1)
    o_ref[0] = idx + base


def _edge_body(nb_ref, cen_ref, w1a_ref, w1b_ref, b1_ref, w2_ref, b2_ref,
               o_ref, *, k, hp):
    nb = nb_ref[...]
    cen = cen_ref[0]
    r, c = cen.shape
    h1n = w1a_ref.shape[1]
    cenr = jnp.broadcast_to(cen[:, None, :], (r, k, c)).reshape(r * k, c)
    e = nb - cenr
    c1 = jnp.dot(cen, w1b_ref[...]) + b1_ref[...]
    c1r = jnp.broadcast_to(c1[:, None, :], (r, k, h1n)).reshape(r * k, h1n)
    h1 = jax.nn.gelu((jnp.dot(e, w1a_ref[...]) + c1r) * _BN)
    h2 = jax.nn.gelu((jnp.dot(h1, w2_ref[...]) + b2_ref[...]) * _BN)
    h2d = h2.shape[1]
    xo = jnp.max(h2.reshape(r, k, h2d), axis=1)
    if hp > h2d:
        xo = jnp.concatenate(
            [xo, jnp.zeros((r, hp - h2d), xo.dtype)], axis=1)
    o_ref[0] = xo


def _pw_body(x1_ref, x2_ref, wa_ref, wb_ref, b1_ref, w2_ref, b2_ref, o_ref):
    h = jax.nn.gelu(
        (jnp.dot(x1_ref[0], wa_ref[...]) + jnp.dot(x2_ref[0], wb_ref[...])
         + b1_ref[...]) * _BN)
    o_ref[0] = jax.nn.gelu((jnp.dot(h, w2_ref[...]) + b2_ref[...]) * _BN)


def _make_sc_gather(n_rows, d, b_total):
    info = plsc.get_sparse_core_info()
    nc, ns = info.num_cores, info.num_subcores
    nw = nc * ns
    assert b_total % (8 * nw) == 0, (b_total, nw)
    b_per_w = b_total // nw
    ch = min(b_per_w, 128)
    assert b_per_w % ch == 0
    n_chunks = b_per_w // ch
    mesh = plsc.VectorSubcoreMesh(core_axis_name="c", subcore_axis_name="s")

    if n_chunks < 2:
        @functools.partial(
            pl.kernel, mesh=mesh,
            out_type=jax.ShapeDtypeStruct((b_total, d), jnp.float32),
            scratch_types=[
                pltpu.VMEM((ch,), jnp.int32),
                pltpu.VMEM((ch, d), jnp.float32),
                pltpu.SemaphoreType.DMA,
            ])
        def gath(table_hbm, idx_hbm, out_hbm, idx_v, rows_v, sem):
            wid = lax.axis_index("s") * nc + lax.axis_index("c")
            base = wid * b_per_w
            pltpu.sync_copy(idx_hbm.at[pl.ds(base, ch)], idx_v)
            pltpu.async_copy(table_hbm.at[idx_v], rows_v, sem).wait()
            pltpu.sync_copy(rows_v, out_hbm.at[pl.ds(base, ch)])
        return gath

    assert n_chunks % 2 == 0

    @functools.partial(
        pl.kernel, mesh=mesh,
        out_type=jax.ShapeDtypeStruct((b_total, d), jnp.float32),
        scratch_types=[
            pltpu.VMEM((b_per_w,), jnp.int32),
            pltpu.VMEM((ch, d), jnp.float32),
            pltpu.VMEM((ch, d), jnp.float32),
            pltpu.SemaphoreType.DMA,
            pltpu.SemaphoreType.DMA,
            pltpu.SemaphoreType.DMA,
            pltpu.SemaphoreType.DMA,
        ])
    def gath(table_hbm, idx_hbm, out_hbm, idx_v, rows0, rows1,
             sg0, sg1, sw0, sw1):
        wid = lax.axis_index("s") * nc + lax.axis_index("c")
        base = wid * b_per_w
        pltpu.sync_copy(idx_hbm.at[pl.ds(base, b_per_w)], idx_v)

        def body(i, carry):
            o0 = 2 * i * ch
            o1 = o0 + ch

            @pl.when(i > 0)
            def _():
                pltpu.make_async_copy(
                    rows0, out_hbm.at[pl.ds(0, ch)], sw0).wait()
                pltpu.make_async_copy(
                    rows1, out_hbm.at[pl.ds(0, ch)], sw1).wait()

            g0 = pltpu.async_copy(
                table_hbm.at[idx_v.at[pl.ds(o0, ch)]], rows0, sg0)
            g1 = pltpu.async_copy(
                table_hbm.at[idx_v.at[pl.ds(o1, ch)]], rows1, sg1)
            g0.wait()
            pltpu.async_copy(rows0, out_hbm.at[pl.ds(base + o0, ch)], sw0)
            g1.wait()
            pltpu.async_copy(rows1, out_hbm.at[pl.ds(base + o1, ch)], sw1)
            return carry

        lax.fori_loop(0, n_chunks // 2, body, 0)
        pltpu.make_async_copy(rows0, out_hbm.at[pl.ds(0, ch)], sw0).wait()
        pltpu.make_async_copy(rows1, out_hbm.at[pl.ds(0, ch)], sw1).wait()

    return gath


def _gather_rows(table, idx):
    n_rows, d = table.shape
    return _make_sc_gather(n_rows, d, idx.shape[0])(table, idx)


def _knn_call(x_n, r_blk):
    b, n, c = x_n.shape
    body = functools.partial(_knn_body, n=n, k=_KNN)
    return pl.pallas_call(
        body,
        grid=(b, n // r_blk),
        in_specs=[
            pl.BlockSpec((1, r_blk, c), lambda i, j: (i, j, 0)),
            pl.BlockSpec((1, n, c), lambda i, j: (i, 0, 0)),
        ],
        out_specs=pl.BlockSpec((1, r_blk, _KNN), lambda i, j: (i, j, 0)),
        out_shape=jax.ShapeDtypeStruct((b, n, _KNN), jnp.int32),
    )(x_n, x_n)


def _edge_call(nb, x_n, w1, b1, w2, b2, r_blk, hp):
    b, n, c = x_n.shape
    h1n = w1.shape[0]
    h2n = w2.shape[0]
    w1a = jnp.transpose(w1[:, :c])
    w1b = jnp.transpose(w1[:, c:])
    if w1b.shape[0] < c:
        w1b = jnp.pad(w1b, ((0, c - w1b.shape[0]), (0, 0)))
    w2t = jnp.transpose(w2)
    nblk = n // r_blk
    body = functools.partial(_edge_body, k=_KNN, hp=hp)
    return pl.pallas_call(
        body,
        grid=(b, nblk),
        in_specs=[
            pl.BlockSpec((r_blk * _KNN, c), lambda i, j: (i * nblk + j, 0)),
            pl.BlockSpec((1, r_blk, c), lambda i, j: (i, j, 0)),
            pl.BlockSpec((c, h1n), lambda i, j: (0, 0)),
            pl.BlockSpec((c, h1n), lambda i, j: (0, 0)),
            pl.BlockSpec((1, h1n), lambda i, j: (0, 0)),
            pl.BlockSpec((h1n, h2n), lambda i, j: (0, 0)),
            pl.BlockSpec((1, h2n), lambda i, j: (0, 0)),
        ],
        out_specs=pl.BlockSpec((1, r_blk, hp), lambda i, j: (i, j, 0)),
        out_shape=jax.ShapeDtypeStruct((b, n, hp), jnp.float32),
    )(nb, x_n, w1a, w1b, b1.reshape(1, h1n), w2t, b2.reshape(1, h2n))


def _pw_call(x1, x2, w1, b1, w2, b2, r_blk, d1_real):
    b, n, c1p = x1.shape
    c2 = x2.shape[2]
    h1n = w1.shape[0]
    h2n = w2.shape[0]
    wa = jnp.transpose(w1[:, :d1_real])
    wa = jnp.pad(wa, ((0, c1p - d1_real), (0, 0)))
    wb = jnp.transpose(w1[:, d1_real:])
    w2t = jnp.transpose(w2)
    nblk = n // r_blk
    return pl.pallas_call(
        _pw_body,
        grid=(b, nblk),
        in_specs=[
            pl.BlockSpec((1, r_blk, c1p), lambda i, j: (i, j, 0)),
            pl.BlockSpec((1, r_blk, c2), lambda i, j: (i, j, 0)),
            pl.BlockSpec((c1p, h1n), lambda i, j: (0, 0)),
            pl.BlockSpec((c2, h1n), lambda i, j: (0, 0)),
            pl.BlockSpec((1, h1n), lambda i, j: (0, 0)),
            pl.BlockSpec((h1n, h2n), lambda i, j: (0, 0)),
            pl.BlockSpec((1, h2n), lambda i, j: (0, 0)),
        ],
        out_specs=pl.BlockSpec((1, r_blk, h2n), lambda i, j: (i, j, 0)),
        out_shape=jax.ShapeDtypeStruct((b, n, h2n), jnp.float32),
    )(x1, x2, wa, wb, b1.reshape(1, h1n), w2t, b2.reshape(1, h2n))


def _gcn(x_n, gp, r_blk):
    b, n, c = x_n.shape
    (w11, b11), (w12, b12) = gp['c1']
    (w21, b21), (w22, b22) = gp['c2']
    (w31, b31), (w32, b32) = gp['c3']
    h2a = w12.shape[0]
    h2a_p = -(-h2a // 128) * 128

    knn_r = min(n, 512)
    idx1 = _knn_call(x_n, knn_r)
    nb1 = _gather_rows(x_n.reshape(b * n, c), idx1.reshape(-1))
    x1 = _edge_call(nb1, x_n, w11, b11, w12, b12, r_blk, h2a_p)

    w21p = jnp.pad(w21, ((0, 0), (0, 2 * h2a_p - w21.shape[1])))
    w21a = jnp.pad(w21[:, :h2a], ((0, 0), (0, h2a_p - h2a)))
    w21b = jnp.pad(w21[:, h2a:], ((0, 0), (0, h2a_p - h2a)))
    w21ab = jnp.concatenate([w21a, w21b], axis=1)
    del w21p
    idx2 = _knn_call(x1, knn_r)
    nb2 = _gather_rows(x1.reshape(b * n, h2a_p), idx2.reshape(-1))
    x2 = _edge_call(nb2, x1, w21ab, b21, w22, b22, r_blk, w22.shape[0])

    return _pw_call(x1, x2, w31, b31, w32, b32, r_blk, h2a)


def kernel(sparse_fea, dense_fea, params):
    p = params
    b, c, s, sp_ = dense_fea.shape
    nd = s * sp_
    ns = sparse_fea.shape[2]

    tw = jnp.transpose(p['d2s_tW'], (2, 0, 1))
    tb = p['d2s_tb'].reshape(c, 1)
    d2 = dense_fea.reshape(b, c, nd)
    z = pl.pallas_call(
        _conv_body,
        grid=(b,),
        in_specs=[
            pl.BlockSpec((1, c, nd), lambda i: (i, 0, 0)),
            pl.BlockSpec((5, c, c), lambda i: (0, 0, 0)),
            pl.BlockSpec((c, 1), lambda i: (0, 0)),
        ],
        out_specs=pl.BlockSpec((1, c, nd), lambda i: (i, 0, 0)),
        out_shape=jax.ShapeDtypeStruct((b, c, nd), jnp.float32),
    )(d2, tw, tb)
    z4 = z.reshape(b, c, s, sp_)

    csp = sparse_fea.shape[1]
    ws = p['d2s_W'][:, :csp]
    wd = p['d2s_W'][:, csp:]
    wdd = p['s2d_W'][:, :c]
    wr = p['s2d_W'][:, c:]
    us, ud = pl.pallas_call(
        _fuse2_body,
        grid=(b,),
        in_specs=[
            pl.BlockSpec((1, c, s, sp_), lambda i: (i, 0, 0, 0)),
            pl.BlockSpec((1, c, s, sp_), lambda i: (i, 0, 0, 0)),
            pl.BlockSpec((1, csp, ns), lambda i: (i, 0, 0)),
            pl.BlockSpec(ws.shape, lambda i: (0, 0)),
            pl.BlockSpec(wd.shape, lambda i: (0, 0)),
            pl.BlockSpec((csp, 1), lambda i: (0, 0)),
            pl.BlockSpec(wdd.shape, lambda i: (0, 0)),
            pl.BlockSpec(wr.shape, lambda i: (0, 0)),
            pl.BlockSpec((c, 1), lambda i: (0, 0)),
        ],
        out_specs=[
            pl.BlockSpec((1, csp, ns), lambda i: (i, 0, 0)),
            pl.BlockSpec((1, c, s, sp_), lambda i: (i, 0, 0, 0)),
        ],
        out_shape=[
            jax.ShapeDtypeStruct((b, csp, ns), jnp.float32),
            jax.ShapeDtypeStruct((b, c, s, sp_), jnp.float32),
        ],
    )(z4, dense_fea, sparse_fea, ws, wd, p['d2s_b'].reshape(csp, 1),
      wdd, wr, p['s2d_b'].reshape(c, 1))

    us_n = jnp.transpose(us, (0, 2, 1))
    xd_n = jnp.transpose(ud.reshape(b, c, nd), (0, 2, 1))

    sp_out = _gcn(us_n, p['sp_gcn'], r_blk=ns)
    bh = b // 2 if b % 2 == 0 else b
    dn_out = jnp.concatenate(
        [_gcn(xd_n[i:i + bh], p['dn_gcn'], r_blk=256)
         for i in range(0, b, bh)], axis=0)

    sparse_out = jnp.transpose(sp_out, (0, 2, 1))
    dense_out = jnp.transpose(dn_out, (0, 2, 1)).reshape(b, -1, s, sp_)
    return sparse_out, dense_out
